# Initial kernel scaffold; baseline (speedup 1.0000x reference)
import jax
import jax.numpy as jnp
from jax.experimental import pallas as pl


def kernel(h, edge_index, coord, edge_attr, We1, be1, We2, be2, Wn1, bn1, Wn2, bn2, Wc1, bc1, Wc2):
    raise NotImplementedError("write your pallas kernel here")


# speedup vs baseline: 3.4040x; 3.4040x over previous
import functools

import jax
import jax.numpy as jnp
from jax import lax
from jax.experimental import pallas as pl
from jax.experimental.pallas import tpu as pltpu
from jax.experimental.pallas import tpu_sc as plsc

F32 = jnp.float32

N = 10000
E = 320000
D = 128
H = 128
DE = 16

NW = 32
EPW = E // NW
CH = 80
ITERS = EPW // CH

BN = 1000
BE = 2000
NPT = 624
NTAIL = N - 16 * NPT

_LINEAR = pltpu.CompilerParams(use_tc_tiling_on_sc=False)
_MESH = dict(core_axis_name="c", subcore_axis_name="s")


def _silu(x):
    return x / (1.0 + jnp.exp(-x))


def _prep_body(h_ref, wa_ref, wb_ref, ta_ref, tb_ref):
    hh = h_ref[...]
    ta_ref[...] = jnp.dot(hh, wa_ref[...], preferred_element_type=F32)
    tb_ref[...] = jnp.dot(hh, wb_ref[...], preferred_element_type=F32)


def _prep(h, wa, wb):
    return pl.pallas_call(
        _prep_body,
        grid=(N // BN,),
        in_specs=[
            pl.BlockSpec((BN, D), lambda i: (i, 0)),
            pl.BlockSpec((D, H), lambda i: (0, 0)),
            pl.BlockSpec((D, H), lambda i: (0, 0)),
        ],
        out_specs=[
            pl.BlockSpec((BN, H), lambda i: (i, 0)),
            pl.BlockSpec((BN, H), lambda i: (i, 0)),
        ],
        out_shape=[
            jax.ShapeDtypeStruct((N, H), F32),
            jax.ShapeDtypeStruct((N, H), F32),
        ],
    )(h, wa, wb)


def _make_gather2(width, params):
    @functools.partial(
        pl.kernel,
        mesh=plsc.VectorSubcoreMesh(**_MESH),
        out_type=[
            jax.ShapeDtypeStruct((E, width), F32),
            jax.ShapeDtypeStruct((E, width), F32),
        ],
        scratch_types=[
            pltpu.VMEM((CH,), jnp.int32),
            pltpu.VMEM((CH,), jnp.int32),
            pltpu.VMEM((CH, width), F32),
            pltpu.VMEM((CH, width), F32),
            pltpu.SemaphoreType.DMA,
            pltpu.SemaphoreType.DMA,
        ],
        compiler_params=params,
    )
    def gather_k(rowi, coli, ta, tb, outa, outb, idxr, idxc, bufa, bufb, sema, semb):
        wid = lax.axis_index("s") * 2 + lax.axis_index("c")

        def body(j, carry):
            base = wid * EPW + j * CH
            pltpu.sync_copy(rowi.at[pl.ds(base, CH)], idxr)
            pltpu.sync_copy(coli.at[pl.ds(base, CH)], idxc)
            ca = pltpu.async_copy(ta.at[idxr], bufa, sema)
            cb = pltpu.async_copy(tb.at[idxc], bufb, semb)
            ca.wait()
            cb.wait()
            pltpu.sync_copy(bufa, outa.at[pl.ds(base, CH)])
            pltpu.sync_copy(bufb, outb.at[pl.ds(base, CH)])
            return carry

        lax.fori_loop(0, ITERS, body, 0)

    return gather_k


_gather_ab = _make_gather2(H, None)
_gather_xy = _make_gather2(16, _LINEAR)


def _edge_body(ga_ref, gb_ref, cr_ref, cc_ref, ea_ref, w1c_ref, b1_ref, wr_ref,
               w2_ref, b2_ref, wc1_ref, bc1_ref, wc2_ref, f_ref, t_ref):
    dc = cr_ref[...] - cc_ref[...]
    radial = jnp.sum(dc * dc, axis=1, keepdims=True)
    pre = (ga_ref[...] + gb_ref[...] + radial * wr_ref[...] +
           jnp.dot(ea_ref[...], w1c_ref[...], preferred_element_type=F32) +
           b1_ref[...])
    m = _silu(pre)
    f = _silu(jnp.dot(m, w2_ref[...], preferred_element_type=F32) + b2_ref[...])
    g1 = _silu(jnp.dot(f, wc1_ref[...], preferred_element_type=F32) + bc1_ref[...])
    gate = jnp.sum(g1 * wc2_ref[...], axis=1, keepdims=True)
    lane = lax.broadcasted_iota(jnp.int32, (1, 16), 1)
    cnt = jnp.where(lane == 3, 1.0, 0.0).astype(F32)
    f_ref[...] = f
    t_ref[...] = dc * gate + cnt


def _edge(ga, gb, cr, cc, ea, w1c, b1, wr, w2, b2, wc1, bc1, wc2):
    full = lambda r, c: pl.BlockSpec((r, c), lambda i: (0, 0))
    return pl.pallas_call(
        _edge_body,
        grid=(E // BE,),
        in_specs=[
            pl.BlockSpec((BE, H), lambda i: (i, 0)),
            pl.BlockSpec((BE, H), lambda i: (i, 0)),
            pl.BlockSpec((BE, 16), lambda i: (i, 0)),
            pl.BlockSpec((BE, 16), lambda i: (i, 0)),
            pl.BlockSpec((BE, DE), lambda i: (i, 0)),
            full(DE, H), full(1, H), full(1, H),
            full(H, H), full(1, H),
            full(H, H), full(1, H), full(1, H),
        ],
        out_specs=[
            pl.BlockSpec((BE, H), lambda i: (i, 0)),
            pl.BlockSpec((BE, 16), lambda i: (i, 0)),
        ],
        out_shape=[
            jax.ShapeDtypeStruct((E, H), F32),
            jax.ShapeDtypeStruct((E, 16), F32),
        ],
    )(ga, gb, cr, cc, ea, w1c, b1, wr, w2, b2, wc1, bc1, wc2)


def _make_scatter(width, params):
    @functools.partial(
        pl.kernel,
        mesh=plsc.VectorSubcoreMesh(**_MESH),
        out_type=[jax.ShapeDtypeStruct((2, N, width), F32)],
        scratch_types=[
            pltpu.VMEM((CH,), jnp.int32),
            pltpu.VMEM((CH, width), F32),
            pltpu.VMEM_SHARED((N, width), F32),
        ],
        compiler_params=params,
    )
    def scatter_k(rowi, ft, zz, parts, idxv, buf, shared):
        cid = lax.axis_index("c")
        sid = lax.axis_index("s")
        wid = sid * 2 + cid
        pltpu.sync_copy(zz.at[pl.ds(sid * NPT, NPT)],
                        shared.at[pl.ds(sid * NPT, NPT)])

        @pl.when(sid == 15)
        def _():
            pltpu.sync_copy(zz.at[pl.ds(16 * NPT, NTAIL)],
                            shared.at[pl.ds(16 * NPT, NTAIL)])

        plsc.subcore_barrier()

        def body(j, carry):
            base = wid * EPW + j * CH
            pltpu.sync_copy(rowi.at[pl.ds(base, CH)], idxv)
            pltpu.sync_copy(ft.at[pl.ds(base, CH)], buf)
            pltpu.sync_copy(buf, shared.at[idxv], add=True)
            return carry

        lax.fori_loop(0, ITERS, body, 0)
        plsc.subcore_barrier()
        pltpu.sync_copy(shared.at[pl.ds(sid * NPT, NPT)],
                        parts.at[cid, pl.ds(sid * NPT, NPT)])

        @pl.when(sid == 15)
        def _():
            pltpu.sync_copy(shared.at[pl.ds(16 * NPT, NTAIL)],
                            parts.at[cid, pl.ds(16 * NPT, NTAIL)])

    return scatter_k


_scatter_f = _make_scatter(H, None)
_scatter_t = _make_scatter(16, _LINEAR)


def _node_body(pf_ref, pt_ref, h_ref, cp_ref, wn1a_ref, wn1b_ref, bn1_ref,
               wn2_ref, bn2_ref, ho_ref, co_ref):
    aggh = pf_ref[0] + pf_ref[1]
    aggt = pt_ref[0] + pt_ref[1]
    denom = jnp.maximum(aggt[:, 3:4], 1.0)
    co_ref[...] = cp_ref[...] + aggt / denom
    hh = h_ref[...]
    t = _silu(jnp.dot(hh, wn1a_ref[...], preferred_element_type=F32) +
              jnp.dot(aggh, wn1b_ref[...], preferred_element_type=F32) +
              bn1_ref[...])
    ho_ref[...] = hh + jnp.dot(t, wn2_ref[...], preferred_element_type=F32) + bn2_ref[...]


def _node(pf, pt, h, cp, wn1a, wn1b, bn1, wn2, bn2):
    full = lambda r, c: pl.BlockSpec((r, c), lambda i: (0, 0))
    return pl.pallas_call(
        _node_body,
        grid=(N // BN,),
        in_specs=[
            pl.BlockSpec((2, BN, H), lambda i: (0, i, 0)),
            pl.BlockSpec((2, BN, 16), lambda i: (0, i, 0)),
            pl.BlockSpec((BN, D), lambda i: (i, 0)),
            pl.BlockSpec((BN, 16), lambda i: (i, 0)),
            full(D, H), full(H, H), full(1, H),
            full(H, D), full(1, D),
        ],
        out_specs=[
            pl.BlockSpec((BN, D), lambda i: (i, 0)),
            pl.BlockSpec((BN, 16), lambda i: (i, 0)),
        ],
        out_shape=[
            jax.ShapeDtypeStruct((N, D), F32),
            jax.ShapeDtypeStruct((N, 16), F32),
        ],
    )(pf, pt, h, cp, wn1a, wn1b, bn1, wn2, bn2)


def kernel(h, edge_index, coord, edge_attr,
           We1, be1, We2, be2, Wn1, bn1, Wn2, bn2, Wc1, bc1, Wc2):
    row = edge_index[0]
    col = edge_index[1]
    cp = jnp.pad(coord, ((0, 0), (0, 13)))

    ta, tb = _prep(h, We1[:H], We1[H:2 * H])
    ga, gb = _gather_ab(row, col, ta, tb)
    cr, cc = _gather_xy(row, col, cp, cp)
    f, t16 = _edge(ga, gb, cr, cc, edge_attr,
                   We1[2 * H + 1:], be1.reshape(1, H), We1[2 * H].reshape(1, H),
                   We2, be2.reshape(1, H), Wc1, bc1.reshape(1, H),
                   Wc2.reshape(1, H))
    pf = _scatter_f(row, f, jnp.zeros((N, H), F32))[0]
    pt = _scatter_t(row, t16, jnp.zeros((N, 16), F32))[0]
    ho, co = _node(pf, pt, h, cp, Wn1[:D], Wn1[D:], bn1.reshape(1, H),
                   Wn2, bn2.reshape(1, D))
    return (ho, co[:, :3], edge_attr)


# speedup vs baseline: 4.6616x; 1.3694x over previous
import functools

import jax
import jax.numpy as jnp
from jax import lax
from jax.experimental import pallas as pl
from jax.experimental.pallas import tpu as pltpu
from jax.experimental.pallas import tpu_sc as plsc

F32 = jnp.float32
BF16 = jnp.bfloat16

N = 10000
E = 320000
D = 128
H = 128
DE = 16

NW = 32
EPW = E // NW
CH = 80
ITERS = EPW // CH

GCH = 40
GITERS = EPW // GCH
NSLOT = 5
OUTER = GITERS // NSLOT

BN = 1000
BE = 2000
NPT = 624
NTAIL = N - 16 * NPT

_LINEAR = pltpu.CompilerParams(use_tc_tiling_on_sc=False)
_MESH = dict(core_axis_name="c", subcore_axis_name="s")


def _silu(x):
    return x / (1.0 + jnp.exp(-x))


def _pack2(lo_bf16, hi_bf16):
    lo = lax.bitcast_convert_type(lo_bf16, jnp.uint16).astype(jnp.uint32)
    hi = lax.bitcast_convert_type(hi_bf16, jnp.uint16).astype(jnp.uint32)
    return lax.bitcast_convert_type(lo | (hi << 16), F32)


def _unpack_lo(x):
    u = lax.bitcast_convert_type(x, jnp.uint32)
    return lax.bitcast_convert_type(
        (u & 0xFFFF).astype(jnp.uint16), BF16).astype(F32)


def _unpack_hi(x):
    u = lax.bitcast_convert_type(x, jnp.uint32)
    return lax.bitcast_convert_type(
        (u >> 16).astype(jnp.uint16), BF16).astype(F32)


def _prep_body(h_ref, cp_ref, wa_ref, wb_ref, ta_ref, tb_ref):
    hh = h_ref[...]
    cpb = cp_ref[...].astype(BF16)
    ta_ref[...] = _pack2(
        jnp.dot(hh, wa_ref[...], preferred_element_type=F32).astype(BF16), cpb)
    tb_ref[...] = _pack2(
        jnp.dot(hh, wb_ref[...], preferred_element_type=F32).astype(BF16), cpb)


def _prep(h, cp128, wa, wb):
    return pl.pallas_call(
        _prep_body,
        grid=(N // BN,),
        in_specs=[
            pl.BlockSpec((BN, D), lambda i: (i, 0)),
            pl.BlockSpec((BN, 128), lambda i: (i, 0)),
            pl.BlockSpec((D, H), lambda i: (0, 0)),
            pl.BlockSpec((D, H), lambda i: (0, 0)),
        ],
        out_specs=[
            pl.BlockSpec((BN, 128), lambda i: (i, 0)),
            pl.BlockSpec((BN, 128), lambda i: (i, 0)),
        ],
        out_shape=[
            jax.ShapeDtypeStruct((N, 128), F32),
            jax.ShapeDtypeStruct((N, 128), F32),
        ],
    )(h, cp128, wa, wb)


def _make_gather():
    @functools.partial(
        pl.kernel,
        mesh=plsc.VectorSubcoreMesh(**_MESH),
        out_type=[
            jax.ShapeDtypeStruct((E, 128), F32),
            jax.ShapeDtypeStruct((E, 128), F32),
        ],
        scratch_types=(
            [pltpu.VMEM((EPW,), jnp.int32)] * 2 +
            [pltpu.VMEM((GCH, 128), F32)] * (2 * NSLOT) +
            [pltpu.SemaphoreType.DMA] * (4 * NSLOT)
        ),
    )
    def gather_k(rowi, coli, ta, tb, outa, outb, idxr, idxc, *rest):
        bufa = rest[0:NSLOT]
        bufb = rest[NSLOT:2 * NSLOT]
        gsa = rest[2 * NSLOT:3 * NSLOT]
        gsb = rest[3 * NSLOT:4 * NSLOT]
        wsa = rest[4 * NSLOT:5 * NSLOT]
        wsb = rest[5 * NSLOT:6 * NSLOT]
        wid = lax.axis_index("s") * 2 + lax.axis_index("c")
        base = wid * EPW
        pltpu.sync_copy(rowi.at[pl.ds(base, EPW)], idxr)
        pltpu.sync_copy(coli.at[pl.ds(base, EPW)], idxc)

        def fire(s, c):
            pltpu.async_copy(ta.at[idxr.at[pl.ds(c * GCH, GCH)]], bufa[s], gsa[s])
            pltpu.async_copy(tb.at[idxc.at[pl.ds(c * GCH, GCH)]], bufb[s], gsb[s])

        for s in range(NSLOT):
            fire(s, s)

        def outer(k, carry):
            c0 = k * NSLOT
            for s in range(NSLOT):
                c = c0 + s
                pltpu.make_async_copy(
                    ta.at[idxr.at[pl.ds(c * GCH, GCH)]], bufa[s], gsa[s]).wait()
                pltpu.make_async_copy(
                    tb.at[idxc.at[pl.ds(c * GCH, GCH)]], bufb[s], gsb[s]).wait()
                pltpu.async_copy(bufa[s], outa.at[pl.ds(base + c * GCH, GCH)], wsa[s])
                pltpu.async_copy(bufb[s], outb.at[pl.ds(base + c * GCH, GCH)], wsb[s])
            for s in range(NSLOT):
                @pl.when(k < OUTER - 1)
                def _():
                    c2 = c0 + NSLOT + s
                    pltpu.make_async_copy(
                        bufa[s], outa.at[pl.ds(base + (c2 - NSLOT) * GCH, GCH)],
                        wsa[s]).wait()
                    pltpu.make_async_copy(
                        bufb[s], outb.at[pl.ds(base + (c2 - NSLOT) * GCH, GCH)],
                        wsb[s]).wait()
                    fire(s, c2)
            return carry

        lax.fori_loop(0, OUTER, outer, 0)
        for s in range(NSLOT):
            c = (OUTER - 1) * NSLOT + s
            pltpu.make_async_copy(
                bufa[s], outa.at[pl.ds(base + c * GCH, GCH)], wsa[s]).wait()
            pltpu.make_async_copy(
                bufb[s], outb.at[pl.ds(base + c * GCH, GCH)], wsb[s]).wait()

    return gather_k


_gather_ab = _make_gather()


def _edge_body(ga_ref, gb_ref, ea_ref, w1c_ref, b1_ref, wr_ref,
               w2_ref, b2_ref, wc1_ref, bc1_ref, wc2_ref, sel_ref, f_ref, t_ref):
    ga = ga_ref[...]
    gb = gb_ref[...]
    dc = _unpack_hi(ga) - _unpack_hi(gb)
    radial = jnp.sum(dc * dc, axis=1, keepdims=True)
    pre = (_unpack_lo(ga) + _unpack_lo(gb) +
           radial * wr_ref[...] +
           jnp.dot(ea_ref[...], w1c_ref[...], preferred_element_type=F32) +
           b1_ref[...])
    m = _silu(pre)
    f = _silu(jnp.dot(m, w2_ref[...], preferred_element_type=F32) + b2_ref[...])
    g1 = _silu(jnp.dot(f, wc1_ref[...], preferred_element_type=F32) + bc1_ref[...])
    gate = jnp.sum(g1 * wc2_ref[...], axis=1, keepdims=True)
    lane = lax.broadcasted_iota(jnp.int32, (1, 16), 1)
    cnt = jnp.where(lane == 3, 1.0, 0.0).astype(F32)
    f_ref[...] = f
    t_ref[...] = jnp.dot(dc * gate, sel_ref[...],
                         preferred_element_type=F32) + cnt


def _edge(ga, gb, ea, w1c, b1, wr, w2, b2, wc1, bc1, wc2, sel):
    full = lambda r, c: pl.BlockSpec((r, c), lambda i: (0, 0))
    gsp = pl.BlockSpec((BE, 128), lambda i: (i, 0))
    return pl.pallas_call(
        _edge_body,
        grid=(E // BE,),
        in_specs=[
            gsp, gsp,
            pl.BlockSpec((BE, DE), lambda i: (i, 0)),
            full(DE, H), full(1, H), full(1, H),
            full(H, H), full(1, H),
            full(H, H), full(1, H), full(1, H),
            full(H, 16),
        ],
        out_specs=[
            pl.BlockSpec((BE, H), lambda i: (i, 0)),
            pl.BlockSpec((BE, 16), lambda i: (i, 0)),
        ],
        out_shape=[
            jax.ShapeDtypeStruct((E, H), F32),
            jax.ShapeDtypeStruct((E, 16), F32),
        ],
    )(ga, gb, ea, w1c, b1, wr, w2, b2, wc1, bc1, wc2, sel)


def _make_scatter(width, params):
    @functools.partial(
        pl.kernel,
        mesh=plsc.VectorSubcoreMesh(**_MESH),
        out_type=[jax.ShapeDtypeStruct((2, N, width), F32)],
        scratch_types=[
            pltpu.VMEM((CH,), jnp.int32),
            pltpu.VMEM((CH, width), F32),
            pltpu.VMEM_SHARED((N, width), F32),
        ],
        compiler_params=params,
    )
    def scatter_k(rowi, ft, zz, parts, idxv, buf, shared):
        cid = lax.axis_index("c")
        sid = lax.axis_index("s")
        wid = sid * 2 + cid
        pltpu.sync_copy(zz.at[pl.ds(sid * NPT, NPT)],
                        shared.at[pl.ds(sid * NPT, NPT)])

        @pl.when(sid == 15)
        def _():
            pltpu.sync_copy(zz.at[pl.ds(16 * NPT, NTAIL)],
                            shared.at[pl.ds(16 * NPT, NTAIL)])

        plsc.subcore_barrier()

        def body(j, carry):
            base = wid * EPW + j * CH
            pltpu.sync_copy(rowi.at[pl.ds(base, CH)], idxv)
            pltpu.sync_copy(ft.at[pl.ds(base, CH)], buf)
            pltpu.sync_copy(buf, shared.at[idxv], add=True)
            return carry

        lax.fori_loop(0, ITERS, body, 0)
        plsc.subcore_barrier()
        pltpu.sync_copy(shared.at[pl.ds(sid * NPT, NPT)],
                        parts.at[cid, pl.ds(sid * NPT, NPT)])

        @pl.when(sid == 15)
        def _():
            pltpu.sync_copy(shared.at[pl.ds(16 * NPT, NTAIL)],
                            parts.at[cid, pl.ds(16 * NPT, NTAIL)])

    return scatter_k


_scatter_f = _make_scatter(H, None)
_scatter_t = _make_scatter(16, _LINEAR)


def _node_body(pf_ref, pt_ref, h_ref, cp_ref, wn1a_ref, wn1b_ref, bn1_ref,
               wn2_ref, bn2_ref, ho_ref, co_ref):
    aggh = pf_ref[0] + pf_ref[1]
    aggt = pt_ref[0] + pt_ref[1]
    denom = jnp.maximum(aggt[:, 3:4], 1.0)
    co_ref[...] = cp_ref[...] + aggt / denom
    hh = h_ref[...]
    t = _silu(jnp.dot(hh, wn1a_ref[...], preferred_element_type=F32) +
              jnp.dot(aggh, wn1b_ref[...], preferred_element_type=F32) +
              bn1_ref[...])
    ho_ref[...] = hh + jnp.dot(t, wn2_ref[...], preferred_element_type=F32) + bn2_ref[...]


def _node(pf, pt, h, cp, wn1a, wn1b, bn1, wn2, bn2):
    full = lambda r, c: pl.BlockSpec((r, c), lambda i: (0, 0))
    return pl.pallas_call(
        _node_body,
        grid=(N // BN,),
        in_specs=[
            pl.BlockSpec((2, BN, H), lambda i: (0, i, 0)),
            pl.BlockSpec((2, BN, 16), lambda i: (0, i, 0)),
            pl.BlockSpec((BN, D), lambda i: (i, 0)),
            pl.BlockSpec((BN, 16), lambda i: (i, 0)),
            full(D, H), full(H, H), full(1, H),
            full(H, D), full(1, D),
        ],
        out_specs=[
            pl.BlockSpec((BN, D), lambda i: (i, 0)),
            pl.BlockSpec((BN, 16), lambda i: (i, 0)),
        ],
        out_shape=[
            jax.ShapeDtypeStruct((N, D), F32),
            jax.ShapeDtypeStruct((N, 16), F32),
        ],
    )(pf, pt, h, cp, wn1a, wn1b, bn1, wn2, bn2)


def kernel(h, edge_index, coord, edge_attr,
           We1, be1, We2, be2, Wn1, bn1, Wn2, bn2, Wc1, bc1, Wc2):
    row = edge_index[0]
    col = edge_index[1]
    cp = jnp.pad(coord, ((0, 0), (0, 13)))
    cp128 = jnp.pad(coord, ((0, 0), (0, 125)))
    sel = jnp.eye(H, 16, dtype=F32)

    ta, tb = _prep(h, cp128, We1[:H], We1[H:2 * H])
    ga, gb = _gather_ab(row, col, ta, tb)
    f, t16 = _edge(ga, gb, edge_attr,
                   We1[2 * H + 1:], be1.reshape(1, H), We1[2 * H].reshape(1, H),
                   We2, be2.reshape(1, H), Wc1, bc1.reshape(1, H),
                   Wc2.reshape(1, H), sel)
    pf = _scatter_f(row, f, jnp.zeros((N, H), F32))[0]
    pt = _scatter_t(row, t16, jnp.zeros((N, 16), F32))[0]
    ho, co = _node(pf, pt, h, cp, Wn1[:D], Wn1[D:], bn1.reshape(1, H),
                   Wn2, bn2.reshape(1, D))
    return (ho, co[:, :3], edge_attr)


# speedup vs baseline: 5.1857x; 1.1124x over previous
import functools

import jax
import jax.numpy as jnp
from jax import lax
from jax.experimental import pallas as pl
from jax.experimental.pallas import tpu as pltpu
from jax.experimental.pallas import tpu_sc as plsc

F32 = jnp.float32
BF16 = jnp.bfloat16

N = 10000
E = 320000
D = 128
H = 128
DE = 16

NW = 32
EPW = E // NW
CH = 80
ITERS = EPW // CH

GCH = 40
GITERS = EPW // GCH
NSLOT = 5
SSLOT = 2
SCH = 40
SITERS = EPW // SCH
OUTER = GITERS // NSLOT

BN = 1000
BE = 2000
NPT = 624
NTAIL = N - 16 * NPT

_LINEAR = pltpu.CompilerParams(use_tc_tiling_on_sc=False)
_MESH = dict(core_axis_name="c", subcore_axis_name="s")


def _silu(x):
    return x / (1.0 + jnp.exp(-x))


def _pack2(lo_bf16, hi_bf16):
    lo = lax.bitcast_convert_type(lo_bf16, jnp.uint16).astype(jnp.uint32)
    hi = lax.bitcast_convert_type(hi_bf16, jnp.uint16).astype(jnp.uint32)
    return lax.bitcast_convert_type(lo | (hi << 16), F32)


def _unpack_lo(x):
    u = lax.bitcast_convert_type(x, jnp.uint32)
    return lax.bitcast_convert_type(
        (u & 0xFFFF).astype(jnp.uint16), BF16).astype(F32)


def _unpack_hi(x):
    u = lax.bitcast_convert_type(x, jnp.uint32)
    return lax.bitcast_convert_type(
        (u >> 16).astype(jnp.uint16), BF16).astype(F32)


def _prep_body(h_ref, cp_ref, wa_ref, wb_ref, ta_ref, tb_ref):
    hh = h_ref[...]
    cpb = cp_ref[...].astype(BF16)
    ta_ref[...] = _pack2(
        jnp.dot(hh, wa_ref[...], preferred_element_type=F32).astype(BF16), cpb)
    tb_ref[...] = _pack2(
        jnp.dot(hh, wb_ref[...], preferred_element_type=F32).astype(BF16), cpb)


def _prep(h, cp128, wa, wb):
    return pl.pallas_call(
        _prep_body,
        grid=(N // BN,),
        in_specs=[
            pl.BlockSpec((BN, D), lambda i: (i, 0)),
            pl.BlockSpec((BN, 128), lambda i: (i, 0)),
            pl.BlockSpec((D, H), lambda i: (0, 0)),
            pl.BlockSpec((D, H), lambda i: (0, 0)),
        ],
        out_specs=[
            pl.BlockSpec((BN, 128), lambda i: (i, 0)),
            pl.BlockSpec((BN, 128), lambda i: (i, 0)),
        ],
        out_shape=[
            jax.ShapeDtypeStruct((N, 128), F32),
            jax.ShapeDtypeStruct((N, 128), F32),
        ],
    )(h, cp128, wa, wb)


def _make_gather():
    @functools.partial(
        pl.kernel,
        mesh=plsc.VectorSubcoreMesh(**_MESH),
        out_type=[
            jax.ShapeDtypeStruct((E, 128), F32),
            jax.ShapeDtypeStruct((E, 128), F32),
        ],
        scratch_types=(
            [pltpu.VMEM((EPW,), jnp.int32)] * 2 +
            [pltpu.VMEM((GCH, 128), F32)] * (2 * NSLOT) +
            [pltpu.SemaphoreType.DMA] * (4 * NSLOT)
        ),
    )
    def gather_k(rowi, coli, ta, tb, outa, outb, idxr, idxc, *rest):
        bufa = rest[0:NSLOT]
        bufb = rest[NSLOT:2 * NSLOT]
        gsa = rest[2 * NSLOT:3 * NSLOT]
        gsb = rest[3 * NSLOT:4 * NSLOT]
        wsa = rest[4 * NSLOT:5 * NSLOT]
        wsb = rest[5 * NSLOT:6 * NSLOT]
        wid = lax.axis_index("s") * 2 + lax.axis_index("c")
        base = wid * EPW
        pltpu.sync_copy(rowi.at[pl.ds(base, EPW)], idxr)
        pltpu.sync_copy(coli.at[pl.ds(base, EPW)], idxc)

        def fire(s, c):
            pltpu.async_copy(ta.at[idxr.at[pl.ds(c * GCH, GCH)]], bufa[s], gsa[s])
            pltpu.async_copy(tb.at[idxc.at[pl.ds(c * GCH, GCH)]], bufb[s], gsb[s])

        for s in range(NSLOT):
            fire(s, s)

        def outer(k, carry):
            c0 = k * NSLOT
            for s in range(NSLOT):
                c = c0 + s
                pltpu.make_async_copy(
                    ta.at[idxr.at[pl.ds(c * GCH, GCH)]], bufa[s], gsa[s]).wait()
                pltpu.make_async_copy(
                    tb.at[idxc.at[pl.ds(c * GCH, GCH)]], bufb[s], gsb[s]).wait()
                pltpu.async_copy(bufa[s], outa.at[pl.ds(base + c * GCH, GCH)], wsa[s])
                pltpu.async_copy(bufb[s], outb.at[pl.ds(base + c * GCH, GCH)], wsb[s])
            for s in range(NSLOT):
                @pl.when(k < OUTER - 1)
                def _():
                    c2 = c0 + NSLOT + s
                    pltpu.make_async_copy(
                        bufa[s], outa.at[pl.ds(base + (c2 - NSLOT) * GCH, GCH)],
                        wsa[s]).wait()
                    pltpu.make_async_copy(
                        bufb[s], outb.at[pl.ds(base + (c2 - NSLOT) * GCH, GCH)],
                        wsb[s]).wait()
                    fire(s, c2)
            return carry

        lax.fori_loop(0, OUTER, outer, 0)
        for s in range(NSLOT):
            c = (OUTER - 1) * NSLOT + s
            pltpu.make_async_copy(
                bufa[s], outa.at[pl.ds(base + c * GCH, GCH)], wsa[s]).wait()
            pltpu.make_async_copy(
                bufb[s], outb.at[pl.ds(base + c * GCH, GCH)], wsb[s]).wait()

    return gather_k


_gather_ab = _make_gather()


def _edge_body(ga_ref, gb_ref, ea_ref, w1c_ref, b1_ref, wr_ref,
               w2_ref, b2_ref, wc1_ref, bc1_ref, wc2_ref, sel_ref, f_ref, t_ref):
    ga = ga_ref[...]
    gb = gb_ref[...]
    dc = _unpack_hi(ga) - _unpack_hi(gb)
    radial = jnp.sum(dc * dc, axis=1, keepdims=True)
    pre = (_unpack_lo(ga) + _unpack_lo(gb) +
           radial * wr_ref[...] +
           jnp.dot(ea_ref[...], w1c_ref[...], preferred_element_type=F32) +
           b1_ref[...])
    m = _silu(pre)
    f = _silu(jnp.dot(m, w2_ref[...], preferred_element_type=F32) + b2_ref[...])
    g1 = _silu(jnp.dot(f, wc1_ref[...], preferred_element_type=F32) + bc1_ref[...])
    gate = jnp.sum(g1 * wc2_ref[...], axis=1, keepdims=True)
    lane = lax.broadcasted_iota(jnp.int32, (1, 16), 1)
    cnt = jnp.where(lane == 3, 1.0, 0.0).astype(F32)
    f_ref[...] = f
    t_ref[...] = jnp.dot(dc * gate, sel_ref[...],
                         preferred_element_type=F32) + cnt


def _edge(ga, gb, ea, w1c, b1, wr, w2, b2, wc1, bc1, wc2, sel):
    full = lambda r, c: pl.BlockSpec((r, c), lambda i: (0, 0))
    gsp = pl.BlockSpec((BE, 128), lambda i: (i, 0))
    return pl.pallas_call(
        _edge_body,
        grid=(E // BE,),
        in_specs=[
            gsp, gsp,
            pl.BlockSpec((BE, DE), lambda i: (i, 0)),
            full(DE, H), full(1, H), full(1, H),
            full(H, H), full(1, H),
            full(H, H), full(1, H), full(1, H),
            full(H, 16),
        ],
        out_specs=[
            pl.BlockSpec((BE, H), lambda i: (i, 0)),
            pl.BlockSpec((BE, 16), lambda i: (i, 0)),
        ],
        out_shape=[
            jax.ShapeDtypeStruct((E, H), F32),
            jax.ShapeDtypeStruct((E, 16), F32),
        ],
    )(ga, gb, ea, w1c, b1, wr, w2, b2, wc1, bc1, wc2, sel)


def _make_scatter(width, params):
    @functools.partial(
        pl.kernel,
        mesh=plsc.VectorSubcoreMesh(**_MESH),
        out_type=[jax.ShapeDtypeStruct((2, N, width), F32)],
        scratch_types=(
            [pltpu.VMEM((SITERS, SCH), jnp.int32)] +
            [pltpu.VMEM((SCH, width), F32)] * SSLOT +
            [pltpu.SemaphoreType.DMA] * (2 * SSLOT) +
            [pltpu.VMEM_SHARED((N, width), F32)]
        ),
        compiler_params=params,
    )
    def scatter_k(rowi3, ft, zz, parts, idx2, *rest):
        buf = rest[0:SSLOT]
        lsem = rest[SSLOT:2 * SSLOT]
        ssem = rest[2 * SSLOT:3 * SSLOT]
        shared = rest[3 * SSLOT]
        cid = lax.axis_index("c")
        sid = lax.axis_index("s")
        wid = sid * 2 + cid
        base = wid * EPW
        pltpu.sync_copy(rowi3.at[wid], idx2)
        pltpu.sync_copy(zz.at[pl.ds(sid * NPT, NPT)],
                        shared.at[pl.ds(sid * NPT, NPT)])

        @pl.when(sid == 15)
        def _():
            pltpu.sync_copy(zz.at[pl.ds(16 * NPT, NTAIL)],
                            shared.at[pl.ds(16 * NPT, NTAIL)])

        plsc.subcore_barrier()

        def fire_load(s, c):
            pltpu.async_copy(ft.at[pl.ds(base + c * SCH, SCH)], buf[s], lsem[s])

        for s in range(SSLOT):
            fire_load(s, s)

        def outer(k, carry):
            c0 = k * SSLOT
            for s in range(SSLOT):
                c = c0 + s
                pltpu.make_async_copy(
                    ft.at[pl.ds(base + c * SCH, SCH)], buf[s], lsem[s]).wait()
                pltpu.async_copy(buf[s], shared.at[idx2.at[c]], ssem[s],
                                 add=True)
            for s in range(SSLOT):
                @pl.when(k < (SITERS // SSLOT) - 1)
                def _():
                    c2 = c0 + SSLOT + s
                    pltpu.make_async_copy(
                        ft.at[pl.ds(base, SCH)], buf[s], ssem[s]).wait()
                    fire_load(s, c2)
            return carry

        lax.fori_loop(0, SITERS // SSLOT, outer, 0)
        for s in range(SSLOT):
            pltpu.make_async_copy(
                ft.at[pl.ds(base, SCH)], buf[s], ssem[s]).wait()
        plsc.subcore_barrier()
        pltpu.sync_copy(shared.at[pl.ds(sid * NPT, NPT)],
                        parts.at[cid, pl.ds(sid * NPT, NPT)])

        @pl.when(sid == 15)
        def _():
            pltpu.sync_copy(shared.at[pl.ds(16 * NPT, NTAIL)],
                            parts.at[cid, pl.ds(16 * NPT, NTAIL)])

    return scatter_k


_scatter_f = _make_scatter(H, None)
_scatter_t = _make_scatter(16, _LINEAR)


def _node_body(pf_ref, pt_ref, h_ref, cp_ref, wn1a_ref, wn1b_ref, bn1_ref,
               wn2_ref, bn2_ref, ho_ref, co_ref):
    aggh = pf_ref[0] + pf_ref[1]
    aggt = pt_ref[0] + pt_ref[1]
    denom = jnp.maximum(aggt[:, 3:4], 1.0)
    co_ref[...] = cp_ref[...] + aggt / denom
    hh = h_ref[...]
    t = _silu(jnp.dot(hh, wn1a_ref[...], preferred_element_type=F32) +
              jnp.dot(aggh, wn1b_ref[...], preferred_element_type=F32) +
              bn1_ref[...])
    ho_ref[...] = hh + jnp.dot(t, wn2_ref[...], preferred_element_type=F32) + bn2_ref[...]


def _node(pf, pt, h, cp, wn1a, wn1b, bn1, wn2, bn2):
    full = lambda r, c: pl.BlockSpec((r, c), lambda i: (0, 0))
    return pl.pallas_call(
        _node_body,
        grid=(N // BN,),
        in_specs=[
            pl.BlockSpec((2, BN, H), lambda i: (0, i, 0)),
            pl.BlockSpec((2, BN, 16), lambda i: (0, i, 0)),
            pl.BlockSpec((BN, D), lambda i: (i, 0)),
            pl.BlockSpec((BN, 16), lambda i: (i, 0)),
            full(D, H), full(H, H), full(1, H),
            full(H, D), full(1, D),
        ],
        out_specs=[
            pl.BlockSpec((BN, D), lambda i: (i, 0)),
            pl.BlockSpec((BN, 16), lambda i: (i, 0)),
        ],
        out_shape=[
            jax.ShapeDtypeStruct((N, D), F32),
            jax.ShapeDtypeStruct((N, 16), F32),
        ],
    )(pf, pt, h, cp, wn1a, wn1b, bn1, wn2, bn2)


def kernel(h, edge_index, coord, edge_attr,
           We1, be1, We2, be2, Wn1, bn1, Wn2, bn2, Wc1, bc1, Wc2):
    row = edge_index[0]
    col = edge_index[1]
    cp = jnp.pad(coord, ((0, 0), (0, 13)))
    cp128 = jnp.pad(coord, ((0, 0), (0, 125)))
    sel = jnp.eye(H, 16, dtype=F32)

    ta, tb = _prep(h, cp128, We1[:H], We1[H:2 * H])
    ga, gb = _gather_ab(row, col, ta, tb)
    f, t16 = _edge(ga, gb, edge_attr,
                   We1[2 * H + 1:], be1.reshape(1, H), We1[2 * H].reshape(1, H),
                   We2, be2.reshape(1, H), Wc1, bc1.reshape(1, H),
                   Wc2.reshape(1, H), sel)
    row3 = row.reshape(NW, SITERS, SCH)
    pf = _scatter_f(row3, f, jnp.zeros((N, H), F32))[0]
    pt = _scatter_t(row3, t16, jnp.zeros((N, 16), F32))[0]
    ho, co = _node(pf, pt, h, cp, Wn1[:D], Wn1[D:], bn1.reshape(1, H),
                   Wn2, bn2.reshape(1, D))
    return (ho, co[:, :3], edge_attr)


# speedup vs baseline: 5.2425x; 1.0109x over previous
import functools

import jax
import jax.numpy as jnp
from jax import lax
from jax.experimental import pallas as pl
from jax.experimental.pallas import tpu as pltpu
from jax.experimental.pallas import tpu_sc as plsc

F32 = jnp.float32
BF16 = jnp.bfloat16

N = 10000
E = 320000
D = 128
H = 128
DE = 16

NHALF = 2
E2 = E // NHALF
NW = 32

GCH = 40
NSLOT = 5
SCH = 40
SSLOT = 2

BN = 1000
BE = 2000
NPT = 624
NTAIL = N - 16 * NPT

_LINEAR = pltpu.CompilerParams(use_tc_tiling_on_sc=False)
_MESH = dict(core_axis_name="c", subcore_axis_name="s")


def _silu(x):
    return x / (1.0 + jnp.exp(-x))


def _pack2(lo_bf16, hi_bf16):
    lo = lax.bitcast_convert_type(lo_bf16, jnp.uint16).astype(jnp.uint32)
    hi = lax.bitcast_convert_type(hi_bf16, jnp.uint16).astype(jnp.uint32)
    return lax.bitcast_convert_type(lo | (hi << 16), F32)


def _unpack_lo(x):
    u = lax.bitcast_convert_type(x, jnp.uint32)
    return lax.bitcast_convert_type(
        (u & 0xFFFF).astype(jnp.uint16), BF16).astype(F32)


def _unpack_hi(x):
    u = lax.bitcast_convert_type(x, jnp.uint32)
    return lax.bitcast_convert_type(
        (u >> 16).astype(jnp.uint16), BF16).astype(F32)


def _prep_body(h_ref, cp_ref, wa_ref, wb_ref, ta_ref, tb_ref):
    hh = h_ref[...].astype(BF16)
    cpb = cp_ref[...]
    ta_ref[...] = _pack2(
        jnp.dot(hh, wa_ref[...], preferred_element_type=F32).astype(BF16), cpb)
    tb_ref[...] = _pack2(
        jnp.dot(hh, wb_ref[...], preferred_element_type=F32).astype(BF16), cpb)


def _prep(h, cp128, wa, wb):
    return pl.pallas_call(
        _prep_body,
        grid=(N // BN,),
        in_specs=[
            pl.BlockSpec((BN, D), lambda i: (i, 0)),
            pl.BlockSpec((BN, 128), lambda i: (i, 0)),
            pl.BlockSpec((D, H), lambda i: (0, 0)),
            pl.BlockSpec((D, H), lambda i: (0, 0)),
        ],
        out_specs=[
            pl.BlockSpec((BN, 128), lambda i: (i, 0)),
            pl.BlockSpec((BN, 128), lambda i: (i, 0)),
        ],
        out_shape=[
            jax.ShapeDtypeStruct((N, 128), F32),
            jax.ShapeDtypeStruct((N, 128), F32),
        ],
    )(h, cp128.astype(BF16), wa.astype(BF16), wb.astype(BF16))


def _make_gather(e_tot):
    epw = e_tot // NW
    giters = epw // GCH
    outer_n = giters // NSLOT

    @functools.partial(
        pl.kernel,
        mesh=plsc.VectorSubcoreMesh(**_MESH),
        out_type=[
            jax.ShapeDtypeStruct((e_tot, 128), F32),
            jax.ShapeDtypeStruct((e_tot, 128), F32),
        ],
        scratch_types=(
            [pltpu.VMEM((epw,), jnp.int32)] * 2 +
            [pltpu.VMEM((GCH, 128), F32)] * (2 * NSLOT) +
            [pltpu.SemaphoreType.DMA] * (4 * NSLOT)
        ),
    )
    def gather_k(rowi, coli, ta, tb, outa, outb, idxr, idxc, *rest):
        bufa = rest[0:NSLOT]
        bufb = rest[NSLOT:2 * NSLOT]
        gsa = rest[2 * NSLOT:3 * NSLOT]
        gsb = rest[3 * NSLOT:4 * NSLOT]
        wsa = rest[4 * NSLOT:5 * NSLOT]
        wsb = rest[5 * NSLOT:6 * NSLOT]
        wid = lax.axis_index("s") * 2 + lax.axis_index("c")
        base = wid * epw
        pltpu.sync_copy(rowi.at[pl.ds(base, epw)], idxr)
        pltpu.sync_copy(coli.at[pl.ds(base, epw)], idxc)

        def fire(s, c):
            pltpu.async_copy(ta.at[idxr.at[pl.ds(c * GCH, GCH)]], bufa[s], gsa[s])
            pltpu.async_copy(tb.at[idxc.at[pl.ds(c * GCH, GCH)]], bufb[s], gsb[s])

        for s in range(NSLOT):
            fire(s, s)

        def outer(k, carry):
            c0 = k * NSLOT
            for s in range(NSLOT):
                c = c0 + s
                pltpu.make_async_copy(
                    ta.at[idxr.at[pl.ds(c * GCH, GCH)]], bufa[s], gsa[s]).wait()
                pltpu.make_async_copy(
                    tb.at[idxc.at[pl.ds(c * GCH, GCH)]], bufb[s], gsb[s]).wait()
                pltpu.async_copy(bufa[s], outa.at[pl.ds(base + c * GCH, GCH)], wsa[s])
                pltpu.async_copy(bufb[s], outb.at[pl.ds(base + c * GCH, GCH)], wsb[s])
            for s in range(NSLOT):
                @pl.when(k < outer_n - 1)
                def _():
                    c2 = c0 + NSLOT + s
                    pltpu.make_async_copy(
                        bufa[s], outa.at[pl.ds(base + (c2 - NSLOT) * GCH, GCH)],
                        wsa[s]).wait()
                    pltpu.make_async_copy(
                        bufb[s], outb.at[pl.ds(base + (c2 - NSLOT) * GCH, GCH)],
                        wsb[s]).wait()
                    fire(s, c2)
            return carry

        lax.fori_loop(0, outer_n, outer, 0)
        for s in range(NSLOT):
            c = (outer_n - 1) * NSLOT + s
            pltpu.make_async_copy(
                bufa[s], outa.at[pl.ds(base + c * GCH, GCH)], wsa[s]).wait()
            pltpu.make_async_copy(
                bufb[s], outb.at[pl.ds(base + c * GCH, GCH)], wsb[s]).wait()

    return gather_k


_gather2 = _make_gather(E2)


def _edge_body(ga_ref, gb_ref, ea_ref, w1c_ref, b1_ref, wr_ref,
               w2_ref, b2_ref, wc1_ref, bc1_ref, wc2_ref, sel_ref, f_ref, t_ref):
    ga = ga_ref[...]
    gb = gb_ref[...]
    dc = _unpack_hi(ga) - _unpack_hi(gb)
    radial = jnp.sum(dc * dc, axis=1, keepdims=True)
    pre = (_unpack_lo(ga) + _unpack_lo(gb) +
           radial * wr_ref[...] +
           jnp.dot(ea_ref[...], w1c_ref[...], preferred_element_type=F32) +
           b1_ref[...])
    m = _silu(pre).astype(BF16)
    f = _silu(jnp.dot(m, w2_ref[...], preferred_element_type=F32) + b2_ref[...])
    fb = f.astype(BF16)
    g1 = _silu(jnp.dot(fb, wc1_ref[...], preferred_element_type=F32) + bc1_ref[...])
    gate = jnp.sum(g1 * wc2_ref[...], axis=1, keepdims=True)
    lane = lax.broadcasted_iota(jnp.int32, (1, 16), 1)
    cnt = jnp.where(lane == 3, 1.0, 0.0).astype(F32)
    f_ref[...] = f
    t_ref[...] = jnp.dot(dc * gate, sel_ref[...],
                         preferred_element_type=F32) + cnt


def _edge(ga, gb, ea, w1c, b1, wr, w2, b2, wc1, bc1, wc2, sel):
    full = lambda r, c: pl.BlockSpec((r, c), lambda i: (0, 0))
    gsp = pl.BlockSpec((BE, 128), lambda i: (i, 0))
    return pl.pallas_call(
        _edge_body,
        grid=(E2 // BE,),
        in_specs=[
            gsp, gsp,
            pl.BlockSpec((BE, DE), lambda i: (i, 0)),
            full(DE, H), full(1, H), full(1, H),
            full(H, H), full(1, H),
            full(H, H), full(1, H), full(1, H),
            full(H, 16),
        ],
        out_specs=[
            pl.BlockSpec((BE, H), lambda i: (i, 0)),
            pl.BlockSpec((BE, 16), lambda i: (i, 0)),
        ],
        out_shape=[
            jax.ShapeDtypeStruct((E2, H), F32),
            jax.ShapeDtypeStruct((E2, 16), F32),
        ],
    )(ga, gb, ea, w1c, b1, wr, w2, b2, wc1, bc1, wc2, sel)


def _make_scatter(width, params, e_tot):
    epw = e_tot // NW
    sit = epw // SCH
    ngrp = sit // SSLOT
    rem = sit % SSLOT

    @functools.partial(
        pl.kernel,
        mesh=plsc.VectorSubcoreMesh(**_MESH),
        out_type=[jax.ShapeDtypeStruct((2, N, width), F32)],
        scratch_types=(
            [pltpu.VMEM((sit, SCH), jnp.int32)] +
            [pltpu.VMEM((SCH, width), F32)] * SSLOT +
            [pltpu.SemaphoreType.DMA] * (2 * SSLOT) +
            [pltpu.VMEM_SHARED((N, width), F32)]
        ),
        compiler_params=params,
    )
    def scatter_k(rowi3, ft, zz, parts, idx2, *rest):
        buf = rest[0:SSLOT]
        lsem = rest[SSLOT:2 * SSLOT]
        ssem = rest[2 * SSLOT:3 * SSLOT]
        shared = rest[3 * SSLOT]
        cid = lax.axis_index("c")
        sid = lax.axis_index("s")
        wid = sid * 2 + cid
        base = wid * epw
        pltpu.sync_copy(rowi3.at[wid], idx2)
        pltpu.sync_copy(zz.at[pl.ds(sid * NPT, NPT)],
                        shared.at[pl.ds(sid * NPT, NPT)])

        @pl.when(sid == 15)
        def _():
            pltpu.sync_copy(zz.at[pl.ds(16 * NPT, NTAIL)],
                            shared.at[pl.ds(16 * NPT, NTAIL)])

        plsc.subcore_barrier()

        def fire_load(s, c):
            pltpu.async_copy(ft.at[pl.ds(base + c * SCH, SCH)], buf[s], lsem[s])

        for s in range(SSLOT):
            fire_load(s, s)

        def outer(k, carry):
            c0 = k * SSLOT
            for s in range(SSLOT):
                c = c0 + s
                pltpu.make_async_copy(
                    ft.at[pl.ds(base + c * SCH, SCH)], buf[s], lsem[s]).wait()
                pltpu.async_copy(buf[s], shared.at[idx2.at[c]], ssem[s],
                                 add=True)
            for s in range(SSLOT):
                c2 = c0 + SSLOT + s

                @pl.when(c2 < sit)
                def _():
                    pltpu.make_async_copy(
                        ft.at[pl.ds(base, SCH)], buf[s], ssem[s]).wait()
                    fire_load(s, c2)
            return carry

        lax.fori_loop(0, ngrp, outer, 0)
        for s in range(rem):
            c = ngrp * SSLOT + s
            pltpu.make_async_copy(
                ft.at[pl.ds(base + c * SCH, SCH)], buf[s], lsem[s]).wait()
            pltpu.async_copy(buf[s], shared.at[idx2.at[c]], ssem[s], add=True)
        for s in range(SSLOT):
            pltpu.make_async_copy(
                ft.at[pl.ds(base, SCH)], buf[s], ssem[s]).wait()
        plsc.subcore_barrier()
        pltpu.sync_copy(shared.at[pl.ds(sid * NPT, NPT)],
                        parts.at[cid, pl.ds(sid * NPT, NPT)])

        @pl.when(sid == 15)
        def _():
            pltpu.sync_copy(shared.at[pl.ds(16 * NPT, NTAIL)],
                            parts.at[cid, pl.ds(16 * NPT, NTAIL)])

    return scatter_k


_scatter_f = _make_scatter(H, None, E2)
_scatter_t = _make_scatter(16, _LINEAR, E2)


def _node_body(pf0_ref, pf1_ref, pt0_ref, pt1_ref, h_ref, cp_ref, wn1a_ref,
               wn1b_ref, bn1_ref, wn2_ref, bn2_ref, ho_ref, co_ref):
    aggh = (pf0_ref[0] + pf0_ref[1]) + (pf1_ref[0] + pf1_ref[1])
    aggt = (pt0_ref[0] + pt0_ref[1]) + (pt1_ref[0] + pt1_ref[1])
    denom = jnp.maximum(aggt[:, 3:4], 1.0)
    co_ref[...] = cp_ref[...] + aggt / denom
    hh = h_ref[...]
    t = _silu(jnp.dot(hh, wn1a_ref[...], preferred_element_type=F32) +
              jnp.dot(aggh, wn1b_ref[...], preferred_element_type=F32) +
              bn1_ref[...])
    ho_ref[...] = hh + jnp.dot(t, wn2_ref[...], preferred_element_type=F32) + bn2_ref[...]


def _node(pf0, pf1, pt0, pt1, h, cp, wn1a, wn1b, bn1, wn2, bn2):
    full = lambda r, c: pl.BlockSpec((r, c), lambda i: (0, 0))
    psp = pl.BlockSpec((2, BN, H), lambda i: (0, i, 0))
    tsp = pl.BlockSpec((2, BN, 16), lambda i: (0, i, 0))
    return pl.pallas_call(
        _node_body,
        grid=(N // BN,),
        in_specs=[
            psp, psp, tsp, tsp,
            pl.BlockSpec((BN, D), lambda i: (i, 0)),
            pl.BlockSpec((BN, 16), lambda i: (i, 0)),
            full(D, H), full(H, H), full(1, H),
            full(H, D), full(1, D),
        ],
        out_specs=[
            pl.BlockSpec((BN, D), lambda i: (i, 0)),
            pl.BlockSpec((BN, 16), lambda i: (i, 0)),
        ],
        out_shape=[
            jax.ShapeDtypeStruct((N, D), F32),
            jax.ShapeDtypeStruct((N, 16), F32),
        ],
    )(pf0, pf1, pt0, pt1, h, cp, wn1a, wn1b, bn1, wn2, bn2)


def kernel(h, edge_index, coord, edge_attr,
           We1, be1, We2, be2, Wn1, bn1, Wn2, bn2, Wc1, bc1, Wc2):
    row = edge_index[0]
    col = edge_index[1]
    cp = jnp.pad(coord, ((0, 0), (0, 13)))
    cp128 = jnp.pad(coord, ((0, 0), (0, 125)))
    sel = jnp.eye(H, 16, dtype=F32)

    ta, tb = _prep(h, cp128, We1[:H], We1[H:2 * H])

    w1c = We1[2 * H + 1:]
    b1 = be1.reshape(1, H)
    wr = We1[2 * H].reshape(1, H)
    b2 = be2.reshape(1, H)
    bc1r = bc1.reshape(1, H)
    wc2r = Wc2.reshape(1, H)
    w2b = We2.astype(BF16)
    wc1b = Wc1.astype(BF16)
    zf = jnp.zeros((N, H), F32)
    zt = jnp.zeros((N, 16), F32)

    sit = (E2 // NW) // SCH
    halves = []
    for i in range(NHALF):
        r = lax.slice_in_dim(row, i * E2, (i + 1) * E2)
        c = lax.slice_in_dim(col, i * E2, (i + 1) * E2)
        ea = lax.slice_in_dim(edge_attr, i * E2, (i + 1) * E2)
        ga, gb = _gather2(r, c, ta, tb)
        f, t16 = _edge(ga, gb, ea, w1c, b1, wr, w2b, b2, wc1b, bc1r, wc2r, sel)
        r3 = r.reshape(NW, sit, SCH)
        pf = _scatter_f(r3, f, zf)[0]
        pt = _scatter_t(r3, t16, zt)[0]
        halves.append((pf, pt))

    ho, co = _node(halves[0][0], halves[1][0], halves[0][1], halves[1][1],
                   h, cp, Wn1[:D], Wn1[D:], bn1.reshape(1, H),
                   Wn2, bn2.reshape(1, D))
    return (ho, co[:, :3], edge_attr)
